# baseline (speedup 1.0000x reference)
import jax
import jax.numpy as jnp
from jax import lax
from jax.experimental import pallas as pl
from jax.experimental.pallas import tpu as pltpu
from jax.experimental.pallas import tpu_sc as plsc

VOCAB = 1000000
D = 32
NS = 16
B, L = 4096, 200
NW = 32
BW = B // NW
LANES = 16
NG = BW // LANES
NSB = L // 4
OP = D + 1


def _sc_body(tokens_hbm, values_hbm, cov_hbm, tab_hbm, w_hbm, bias_hbm,
             out_hbm,
             idx4_0, idx4_1, vls4_0, vls4_1,
             cov_v, w_v, bias_v, static_v,
             tab0, tab1, tab2, tab3, ob0, ob1,
             ld_sems, g_sems, st_sems):
    idx4 = [idx4_0, idx4_1]
    vals4 = [vls4_0, vls4_1]
    tab_v = [tab0, tab1, tab2, tab3]
    obuf_v = [ob0, ob1]

    cid = lax.axis_index("c")
    sid = lax.axis_index("s")
    wid = sid * 2 + cid
    b0 = wid * BW

    pltpu.sync_copy(w_hbm, w_v)
    pltpu.sync_copy(bias_hbm, bias_v)
    for k in range(NS):
        pltpu.sync_copy(cov_hbm.at[k, pl.ds(b0, BW)], cov_v.at[k])

    iota = lax.iota(jnp.int32, LANES)
    zeros = jnp.full((LANES,), 0, dtype=jnp.int32)

    def _proj_j(j, _):
        jv = zeros + j
        bias_j = plsc.load_gather(bias_v, [jv])

        def _proj_g(g, _):
            def _proj_k(k, acc):
                wb = plsc.load_gather(w_v, [zeros + k, jv])
                return acc + wb * cov_v[k, pl.ds(g * LANES, LANES)]
            static_v[j, pl.ds(g * LANES, LANES)] = lax.fori_loop(
                0, NS, _proj_k, bias_j)
            return _
        lax.fori_loop(0, NG, _proj_g, None)
        return _
    lax.fori_loop(0, D, _proj_j, None)

    def _load(blk, r):
        pltpu.make_async_copy(
            tokens_hbm.at[pl.ds(blk * 4, 4), pl.ds(b0, BW)], idx4[r],
            ld_sems.at[r]).start()
        pltpu.make_async_copy(
            values_hbm.at[pl.ds(blk * 4, 4), pl.ds(b0, BW)], vals4[r],
            ld_sems.at[r]).start()

    def _wait_load(r):
        pltpu.make_async_copy(
            tokens_hbm.at[pl.ds(0, 4), pl.ds(b0, BW)], idx4[r],
            ld_sems.at[r]).wait()
        pltpu.make_async_copy(
            values_hbm.at[pl.ds(0, 4), pl.ds(b0, BW)], vals4[r],
            ld_sems.at[r]).wait()

    def _gather(r, row, p):
        pltpu.make_async_copy(
            tab_hbm.at[idx4[r].at[row]], tab_v[p], g_sems.at[p]).start()

    def _wait_gather(p):
        pltpu.make_async_copy(
            tab_hbm.at[idx4[0].at[0]], tab_v[p], g_sems.at[p]).wait()

    def _store(l, p):
        pltpu.make_async_copy(
            obuf_v[p], out_hbm.at[l, pl.ds(0, 4), wid], st_sems.at[p]).start()

    def _wait_store(p):
        pltpu.make_async_copy(
            obuf_v[p], out_hbm.at[0, pl.ds(0, 4), wid], st_sems.at[p]).wait()

    def _compute(r, row, p, po):
        def _grp(g, _):
            v = vals4[r][row, pl.ds(g * LANES, LANES)]
            v = jnp.where(v == v, v, jnp.float32(0.0))
            rows = g * LANES + iota
            for j in range(D):
                jv = (iota + j) & (D - 1)
                c16 = plsc.load_gather(tab_v[p], [rows, jv])
                n16 = plsc.load_gather(tab_v[p], [rows, jv + D])
                s16 = plsc.load_gather(static_v, [jv, rows])
                plsc.store_scatter(obuf_v[po], [jv >> 3, jv & 7, rows],
                                   c16 + v * n16 + s16)
            return _
        lax.fori_loop(0, NG, _grp, None)

    _load(0, 0)
    _load(1, 1)
    _wait_load(0)
    _gather(0, 0, 0)
    _gather(0, 1, 1)

    def _super(i, _):
        for ss in range(2):
            s = 2 * i + ss
            for ph in range(4):
                l = 4 * s + ph

                if ph == 2:
                    @pl.when(s <= NSB - 2)
                    def _():
                        _wait_load(1 - ss)

                gslot, grow = (ss, ph + 2) if ph < 2 else (1 - ss, ph - 2)
                if ph < 2:
                    _gather(gslot, grow, (ph + 2) % 4)
                else:
                    @pl.when(s <= NSB - 2)
                    def _():
                        _gather(gslot, grow, (ph + 2) % 4)

                _wait_gather(ph)

                @pl.when(l >= 2)
                def _():
                    _wait_store(ph % 2)

                _compute(ss, ph, ph, ph % 2)
                _store(l, ph % 2)

                if ph == 3:
                    @pl.when(s <= NSB - 3)
                    def _():
                        _load(s + 2, ss)
        return _

    lax.fori_loop(0, NSB // 2, _super, None)
    _wait_store(0)
    _wait_store(1)


@jax.jit
def _embed(tokens, values, covariates, cat_table, num_table, W_static, b_static):
    mesh = plsc.VectorSubcoreMesh(core_axis_name="c", subcore_axis_name="s",
                                  num_cores=2, num_subcores=16)
    f = pl.kernel(
        _sc_body,
        out_type=jax.ShapeDtypeStruct((L, 4, NW, 8, BW), jnp.float32),
        mesh=mesh,
        scratch_types=(
            [pltpu.VMEM((4, BW), jnp.int32) for _ in range(2)]
            + [pltpu.VMEM((4, BW), jnp.float32) for _ in range(2)]
            + [
                pltpu.VMEM((NS, BW), jnp.float32),
                pltpu.VMEM((NS, D), jnp.float32),
                pltpu.VMEM((D,), jnp.float32),
                pltpu.VMEM((D, BW), jnp.float32),
            ]
            + [pltpu.VMEM((BW, 2 * D), jnp.float32) for _ in range(4)]
            + [pltpu.VMEM((4, 8, BW), jnp.float32) for _ in range(2)]
            + [
                pltpu.SemaphoreType.DMA((2,)),
                pltpu.SemaphoreType.DMA((4,)),
                pltpu.SemaphoreType.DMA((2,)),
            ]
        ),
        compiler_params=pltpu.CompilerParams(use_tc_tiling_on_sc=False,
                                             needs_layout_passes=False),
    )
    table = jnp.concatenate([cat_table, num_table], axis=1)
    out5 = f(tokens.T, values.T, covariates.T, table, W_static, b_static)
    return out5.transpose(2, 4, 0, 1, 3).reshape(B, L, D)


def kernel(tokens, values, covariates, cat_table, num_table, W_static, b_static):
    return _embed(tokens.astype(jnp.int32), values, covariates,
                  cat_table, num_table, W_static, b_static)


# speedup vs baseline: 1.2563x; 1.2563x over previous
import jax
import jax.numpy as jnp
from jax import lax
from jax.experimental import pallas as pl
from jax.experimental.pallas import tpu as pltpu
from jax.experimental.pallas import tpu_sc as plsc

VOCAB = 1000000
D = 32
NS = 16
B, L = 4096, 200
NW = 32
BW = B // NW
LANES = 16
NG = BW // LANES
NSB = L // 4
OP = D + 1


def _sc_body(tokens_hbm, values_hbm, cov_hbm, cat_hbm, num_hbm, w_hbm, bias_hbm,
             out_hbm,
             idx4_0, idx4_1, vls4_0, vls4_1,
             cov_v, w_v, bias_v, static_v,
             cat0, cat1, cat2, cat3, num0, num1, num2, num3, ob0, ob1,
             ld_sems, g_sems, st_sems):
    idx4 = [idx4_0, idx4_1]
    vals4 = [vls4_0, vls4_1]
    cat_v = [cat0, cat1, cat2, cat3]
    num_v = [num0, num1, num2, num3]
    obuf_v = [ob0, ob1]

    cid = lax.axis_index("c")
    sid = lax.axis_index("s")
    wid = sid * 2 + cid
    b0 = wid * BW

    pltpu.sync_copy(w_hbm, w_v)
    pltpu.sync_copy(bias_hbm, bias_v)
    for k in range(NS):
        pltpu.sync_copy(cov_hbm.at[k, pl.ds(b0, BW)], cov_v.at[k])

    iota = lax.iota(jnp.int32, LANES)
    zeros = jnp.full((LANES,), 0, dtype=jnp.int32)

    def _proj_j(j, _):
        jv = zeros + j
        bias_j = plsc.load_gather(bias_v, [jv])

        def _proj_g(g, _):
            def _proj_k(k, acc):
                wb = plsc.load_gather(w_v, [zeros + k, jv])
                return acc + wb * cov_v[k, pl.ds(g * LANES, LANES)]
            static_v[j, pl.ds(g * LANES, LANES)] = lax.fori_loop(
                0, NS, _proj_k, bias_j)
            return _
        lax.fori_loop(0, NG, _proj_g, None)
        return _
    lax.fori_loop(0, D, _proj_j, None)

    def _load(blk, r):
        pltpu.make_async_copy(
            tokens_hbm.at[pl.ds(blk * 4, 4), pl.ds(b0, BW)], idx4[r],
            ld_sems.at[r]).start()
        pltpu.make_async_copy(
            values_hbm.at[pl.ds(blk * 4, 4), pl.ds(b0, BW)], vals4[r],
            ld_sems.at[r]).start()

    def _wait_load(r):
        pltpu.make_async_copy(
            tokens_hbm.at[pl.ds(0, 4), pl.ds(b0, BW)], idx4[r],
            ld_sems.at[r]).wait()
        pltpu.make_async_copy(
            values_hbm.at[pl.ds(0, 4), pl.ds(b0, BW)], vals4[r],
            ld_sems.at[r]).wait()

    def _gather(r, row, p):
        pltpu.make_async_copy(
            cat_hbm.at[idx4[r].at[row]], cat_v[p], g_sems.at[p]).start()
        pltpu.make_async_copy(
            num_hbm.at[idx4[r].at[row]], num_v[p], g_sems.at[p]).start()

    def _wait_gather(p):
        pltpu.make_async_copy(
            cat_hbm.at[idx4[0].at[0]], cat_v[p], g_sems.at[p]).wait()
        pltpu.make_async_copy(
            num_hbm.at[idx4[0].at[0]], num_v[p], g_sems.at[p]).wait()

    def _store(l, p):
        pltpu.make_async_copy(
            obuf_v[p], out_hbm.at[l, pl.ds(0, 4), wid], st_sems.at[p]).start()

    def _wait_store(p):
        pltpu.make_async_copy(
            obuf_v[p], out_hbm.at[0, pl.ds(0, 4), wid], st_sems.at[p]).wait()

    def _compute(r, row, p, po):
        def _grp(g, _):
            v = vals4[r][row, pl.ds(g * LANES, LANES)]
            v = jnp.where(v == v, v, jnp.float32(0.0))
            rows = g * LANES + iota
            for j in range(D):
                jv = (iota + j) & (D - 1)
                c16 = plsc.load_gather(cat_v[p], [rows, jv])
                n16 = plsc.load_gather(num_v[p], [rows, jv])
                s16 = plsc.load_gather(static_v, [jv, rows])
                plsc.store_scatter(obuf_v[po], [jv >> 3, jv & 7, rows],
                                   c16 + v * n16 + s16)
            return _
        lax.fori_loop(0, NG, _grp, None)

    _load(0, 0)
    _load(1, 1)
    _wait_load(0)
    _gather(0, 0, 0)
    _gather(0, 1, 1)

    def _super(i, _):
        for ss in range(2):
            s = 2 * i + ss
            for ph in range(4):
                l = 4 * s + ph

                if ph == 2:
                    @pl.when(s <= NSB - 2)
                    def _():
                        _wait_load(1 - ss)

                gslot, grow = (ss, ph + 2) if ph < 2 else (1 - ss, ph - 2)
                if ph < 2:
                    _gather(gslot, grow, (ph + 2) % 4)
                else:
                    @pl.when(s <= NSB - 2)
                    def _():
                        _gather(gslot, grow, (ph + 2) % 4)

                _wait_gather(ph)

                @pl.when(l >= 2)
                def _():
                    _wait_store(ph % 2)

                _compute(ss, ph, ph, ph % 2)
                _store(l, ph % 2)

                if ph == 3:
                    @pl.when(s <= NSB - 3)
                    def _():
                        _load(s + 2, ss)
        return _

    lax.fori_loop(0, NSB // 2, _super, None)
    _wait_store(0)
    _wait_store(1)


@jax.jit
def _embed(tokens, values, covariates, cat_table, num_table, W_static, b_static):
    mesh = plsc.VectorSubcoreMesh(core_axis_name="c", subcore_axis_name="s",
                                  num_cores=2, num_subcores=16)
    f = pl.kernel(
        _sc_body,
        out_type=jax.ShapeDtypeStruct((L, 4, NW, 8, BW), jnp.float32),
        mesh=mesh,
        scratch_types=(
            [pltpu.VMEM((4, BW), jnp.int32) for _ in range(2)]
            + [pltpu.VMEM((4, BW), jnp.float32) for _ in range(2)]
            + [
                pltpu.VMEM((NS, BW), jnp.float32),
                pltpu.VMEM((NS, D), jnp.float32),
                pltpu.VMEM((D,), jnp.float32),
                pltpu.VMEM((D, BW), jnp.float32),
            ]
            + [pltpu.VMEM((BW, D), jnp.float32) for _ in range(8)]
            + [pltpu.VMEM((4, 8, BW), jnp.float32) for _ in range(2)]
            + [
                pltpu.SemaphoreType.DMA((2,)),
                pltpu.SemaphoreType.DMA((4,)),
                pltpu.SemaphoreType.DMA((2,)),
            ]
        ),
        compiler_params=pltpu.CompilerParams(use_tc_tiling_on_sc=False,
                                             needs_layout_passes=False),
    )
    out5 = f(tokens.T, values.T, covariates.T, cat_table, num_table,
             W_static, b_static)
    return out5.transpose(2, 4, 0, 1, 3).reshape(B, L, D)


def kernel(tokens, values, covariates, cat_table, num_table, W_static, b_static):
    return _embed(tokens.astype(jnp.int32), values, covariates,
                  cat_table, num_table, W_static, b_static)


# speedup vs baseline: 1.2763x; 1.0160x over previous
import jax
import jax.numpy as jnp
from jax import lax
from jax.experimental import pallas as pl
from jax.experimental.pallas import tpu as pltpu
from jax.experimental.pallas import tpu_sc as plsc

VOCAB = 1000000
D = 32
NS = 16
B, L = 4096, 200
NW = 32
BW = B // NW
LANES = 16
NG = BW // LANES
NSB = L // 4
OP = BW + 1


def _sc_body(tokens_hbm, values_hbm, cov_hbm, cat_hbm, num_hbm, w_hbm, bias_hbm,
             out_hbm,
             idx4_0, idx4_1, vls4_0, vls4_1,
             cov_v, w_v, bias_v, static_v,
             cat0, cat1, cat2, cat3, num0, num1, num2, num3, ob0, ob1,
             ld_sems, g_sems, st_sems):
    idx4 = [idx4_0, idx4_1]
    vals4 = [vls4_0, vls4_1]
    cat_v = [cat0, cat1, cat2, cat3]
    num_v = [num0, num1, num2, num3]
    obuf_v = [ob0, ob1]

    cid = lax.axis_index("c")
    sid = lax.axis_index("s")
    wid = sid * 2 + cid
    b0 = wid * BW

    pltpu.sync_copy(w_hbm, w_v)
    pltpu.sync_copy(bias_hbm, bias_v)
    for k in range(NS):
        pltpu.sync_copy(cov_hbm.at[k, pl.ds(b0, BW)], cov_v.at[k])

    iota = lax.iota(jnp.int32, LANES)
    zeros = jnp.full((LANES,), 0, dtype=jnp.int32)

    def _proj_j(j, _):
        jv = zeros + j
        bias_j = plsc.load_gather(bias_v, [jv])

        def _proj_g(g, _):
            def _proj_k(k, acc):
                wb = plsc.load_gather(w_v, [zeros + k, jv])
                return acc + wb * cov_v[k, pl.ds(g * LANES, LANES)]
            acc = lax.fori_loop(0, NS, _proj_k, bias_j)
            plsc.store_scatter(static_v, [g * LANES + iota, jv], acc)
            return _
        lax.fori_loop(0, NG, _proj_g, None)
        return _
    lax.fori_loop(0, D, _proj_j, None)

    ddx = [2 * h + iota // 8 for h in range(2)]
    jdx = iota % 8

    def _load(blk, r):
        pltpu.make_async_copy(
            tokens_hbm.at[pl.ds(blk * 4, 4), pl.ds(b0, BW)], idx4[r],
            ld_sems.at[r]).start()
        pltpu.make_async_copy(
            values_hbm.at[pl.ds(blk * 4, 4), pl.ds(b0, BW)], vals4[r],
            ld_sems.at[r]).start()

    def _wait_load(r):
        pltpu.make_async_copy(
            tokens_hbm.at[pl.ds(0, 4), pl.ds(b0, BW)], idx4[r],
            ld_sems.at[r]).wait()
        pltpu.make_async_copy(
            values_hbm.at[pl.ds(0, 4), pl.ds(b0, BW)], vals4[r],
            ld_sems.at[r]).wait()

    def _gather(r, row, p):
        pltpu.make_async_copy(
            cat_hbm.at[idx4[r].at[row]], cat_v[p], g_sems.at[p]).start()
        pltpu.make_async_copy(
            num_hbm.at[idx4[r].at[row]], num_v[p], g_sems.at[p]).start()

    def _wait_gather(p):
        pltpu.make_async_copy(
            cat_hbm.at[idx4[0].at[0]], cat_v[p], g_sems.at[p]).wait()
        pltpu.make_async_copy(
            num_hbm.at[idx4[0].at[0]], num_v[p], g_sems.at[p]).wait()

    def _store(l, p):
        pltpu.make_async_copy(
            obuf_v[p].at[pl.ds(0, 4), pl.ds(0, 8), pl.ds(0, BW)],
            out_hbm.at[l, pl.ds(0, 4), wid], st_sems.at[p]).start()

    def _wait_store(p):
        pltpu.make_async_copy(
            obuf_v[p].at[pl.ds(0, 4), pl.ds(0, 8), pl.ds(0, BW)],
            out_hbm.at[0, pl.ds(0, 4), wid], st_sems.at[p]).wait()

    def _compute(r, row, p, po):
        def _grp(g, _):
            vblk = vals4[r][row, pl.ds(g * LANES, LANES)]
            vblk = jnp.where(vblk == vblk, vblk, jnp.float32(0.0))
            for c in range(LANES):
                tok = g * LANES + c
                vb = jnp.full((LANES,), vblk[c], dtype=jnp.float32)
                cdx = zeros + tok
                for h in range(2):
                    sl = pl.ds(h * LANES, LANES)
                    o16 = (cat_v[p][tok, sl] + vb * num_v[p][tok, sl]
                           + static_v[tok, sl])
                    plsc.store_scatter(obuf_v[po], [ddx[h], jdx, cdx], o16)
            return _
        lax.fori_loop(0, NG, _grp, None)

    _load(0, 0)
    _load(1, 1)
    _wait_load(0)
    _gather(0, 0, 0)
    _gather(0, 1, 1)

    def _super(i, _):
        for ss in range(2):
            s = 2 * i + ss
            for ph in range(4):
                l = 4 * s + ph

                if ph == 2:
                    @pl.when(s <= NSB - 2)
                    def _():
                        _wait_load(1 - ss)

                gslot, grow = (ss, ph + 2) if ph < 2 else (1 - ss, ph - 2)
                if ph < 2:
                    _gather(gslot, grow, (ph + 2) % 4)
                else:
                    @pl.when(s <= NSB - 2)
                    def _():
                        _gather(gslot, grow, (ph + 2) % 4)

                _wait_gather(ph)

                @pl.when(l >= 2)
                def _():
                    _wait_store(ph % 2)

                _compute(ss, ph, ph, ph % 2)
                _store(l, ph % 2)

                if ph == 3:
                    @pl.when(s <= NSB - 3)
                    def _():
                        _load(s + 2, ss)
        return _

    lax.fori_loop(0, NSB // 2, _super, None)
    _wait_store(0)
    _wait_store(1)


@jax.jit
def _embed(tokens, values, covariates, cat_table, num_table, W_static, b_static):
    mesh = plsc.VectorSubcoreMesh(core_axis_name="c", subcore_axis_name="s",
                                  num_cores=2, num_subcores=16)
    f = pl.kernel(
        _sc_body,
        out_type=jax.ShapeDtypeStruct((L, 4, NW, 8, BW), jnp.float32),
        mesh=mesh,
        scratch_types=(
            [pltpu.VMEM((4, BW), jnp.int32) for _ in range(2)]
            + [pltpu.VMEM((4, BW), jnp.float32) for _ in range(2)]
            + [
                pltpu.VMEM((NS, BW), jnp.float32),
                pltpu.VMEM((NS, D), jnp.float32),
                pltpu.VMEM((D,), jnp.float32),
                pltpu.VMEM((BW, D), jnp.float32),
            ]
            + [pltpu.VMEM((BW, D), jnp.float32) for _ in range(8)]
            + [pltpu.VMEM((4, 8, OP), jnp.float32) for _ in range(2)]
            + [
                pltpu.SemaphoreType.DMA((2,)),
                pltpu.SemaphoreType.DMA((4,)),
                pltpu.SemaphoreType.DMA((2,)),
            ]
        ),
        compiler_params=pltpu.CompilerParams(use_tc_tiling_on_sc=False,
                                             needs_layout_passes=False),
    )
    out5 = f(tokens.T, values.T, covariates.T, cat_table, num_table,
             W_static, b_static)
    return out5.transpose(2, 4, 0, 1, 3).reshape(B, L, D)


def kernel(tokens, values, covariates, cat_table, num_table, W_static, b_static):
    return _embed(tokens.astype(jnp.int32), values, covariates,
                  cat_table, num_table, W_static, b_static)
